# baseline (speedup 1.0000x reference)
import functools

import jax
import jax.numpy as jnp
from jax import lax
from jax.experimental import pallas as pl
from jax.experimental.pallas import tpu as pltpu
from jax.experimental.pallas import tpu_sc as plsc

_SENT = -3.0e38
_R = 1000
_FP = 32
_CH = 1000
_C1 = 50000
_C3 = 5000
_F1 = 20


def _seg_scatter_max(seg_ref, rows_ref, out_ref, nrows):
    def body(i, carry):
        prev, cur = carry
        s = seg_ref[0, 0, i]
        row = rows_ref[pl.ds(i, 1), :]
        same = s == prev

        @pl.when(jnp.logical_not(same))
        def _():
            out_ref[pl.ds(prev, 1), :] = jnp.maximum(
                out_ref[pl.ds(prev, 1), :], cur)

        new_cur = jnp.where(same, jnp.maximum(cur, row), row)
        return (s, new_cur)

    s0 = seg_ref[0, 0, 0]
    row0 = rows_ref[pl.ds(0, 1), :]
    prev, cur = lax.fori_loop(1, nrows, body, (s0, row0))
    out_ref[pl.ds(prev, 1), :] = jnp.maximum(out_ref[pl.ds(prev, 1), :], cur)


def _init_fix(out_ref, pid):
    @pl.when(pid == 0)
    def _():
        out_ref[...] = jnp.full(out_ref.shape, _SENT, out_ref.dtype)


def _fixup(out_ref, pid):
    @pl.when(pid == pl.num_programs(0) - 1)
    def _():
        o = out_ref[...]
        out_ref[...] = jnp.where(o == _SENT, 0.0, o)


def _dot(a, b):
    return jnp.dot(a, b, preferred_element_type=jnp.float32)


def _enc1_kernel(seg_ref, x_ref, w1_ref, b1_ref, w2_ref, b2_ref, w3_ref,
                 b3_ref, out_ref, h_ref):
    pid = pl.program_id(0)
    _init_fix(out_ref, pid)
    x = x_ref[...]
    h = jnp.maximum(_dot(x, w1_ref[...]) + b1_ref[...], 0.0)
    h = jnp.maximum(_dot(h, w2_ref[...]) + b2_ref[...], 0.0)
    h = _dot(h, w3_ref[...]) + b3_ref[...]
    pad = jnp.zeros((h.shape[0], _FP - h.shape[1]), h.dtype)
    h_ref[...] = jnp.concatenate([h, pad], axis=1)
    _seg_scatter_max(seg_ref, h_ref, out_ref, x.shape[0])
    _fixup(out_ref, pid)


def _enc1(x, seg, w1, b1, w2, b2, w3, b3, c_out, interpret=False):
    n = x.shape[0]
    full2 = lambda a: pl.BlockSpec(a.shape, lambda i: (0, 0))
    return pl.pallas_call(
        _enc1_kernel,
        grid=(n // _R,),
        in_specs=[
            pl.BlockSpec((1, 1, _R), lambda i: (i, 0, 0), memory_space=pltpu.SMEM),
            pl.BlockSpec((_R, 3), lambda i: (i, 0)),
            full2(w1), full2(b1), full2(w2), full2(b2), full2(w3), full2(b3),
        ],
        out_specs=pl.BlockSpec((c_out, _FP), lambda i: (0, 0)),
        out_shape=jax.ShapeDtypeStruct((c_out, _FP), jnp.float32),
        scratch_shapes=[pltpu.VMEM((_R, _FP), jnp.float32)],
        interpret=interpret,
    )(seg.reshape(n // _R, 1, _R), x, w1, b1, w2, b2, w3, b3)


def _gather_rows(table, idx):
    n = idx.shape[0]
    info = plsc.get_sparse_core_info()
    nw = info.num_cores * info.num_subcores
    bpw = n // nw
    nch = bpw // _CH
    mesh = plsc.VectorSubcoreMesh(core_axis_name="c", subcore_axis_name="s")

    @functools.partial(
        pl.kernel,
        mesh=mesh,
        compiler_params=pltpu.CompilerParams(use_tc_tiling_on_sc=False),
        out_type=jax.ShapeDtypeStruct((n, _FP), jnp.float32),
        scratch_types=[
            pltpu.VMEM((_CH,), jnp.int32),
            pltpu.VMEM((_CH, _FP), jnp.float32),
            pltpu.SemaphoreType.DMA,
        ],
    )
    def k(table_hbm, idx_hbm, out_hbm, idx_v, rows_v, sem):
        wid = lax.axis_index("s") * info.num_cores + lax.axis_index("c")
        base = wid * bpw

        def body(ci, carry):
            off = base + ci * _CH
            pltpu.sync_copy(idx_hbm.at[pl.ds(off, _CH)], idx_v)
            pltpu.async_copy(table_hbm.at[idx_v], rows_v, sem).wait()
            pltpu.sync_copy(rows_v, out_hbm.at[pl.ds(off, _CH)])
            return carry

        lax.fori_loop(0, nch, body, 0)

    return k(table, idx)


def _enc2_kernel(seg_ref, x_ref, fm_ref, a1_ref, av1_ref, a2_ref, av2_ref,
                 b1r_ref, b1f_ref, bb1_ref, out_ref, row_ref):
    pid = pl.program_id(0)
    _init_fix(out_ref, pid)
    x = x_ref[...]
    r = jnp.maximum(_dot(x, a1_ref[...]) + av1_ref[...], 0.0)
    r = jnp.maximum(_dot(r, a2_ref[...]) + av2_ref[...], 0.0)
    fm = fm_ref[...][:, :_F1]
    e = jnp.maximum(_dot(r, b1r_ref[...]) + _dot(fm, b1f_ref[...])
                    + bb1_ref[...], 0.0)
    row_ref[...] = jnp.concatenate([e, fm], axis=1)
    _seg_scatter_max(seg_ref, row_ref, out_ref, x.shape[0])
    _fixup(out_ref, pid)


def _enc2(x, fm, seg, a1, av1, a2, av2, b1r, b1f, bb1, c_out,
          interpret=False):
    n = x.shape[0]
    w50 = b1r.shape[1] + _F1
    full2 = lambda a: pl.BlockSpec(a.shape, lambda i: (0, 0))
    return pl.pallas_call(
        _enc2_kernel,
        grid=(n // _R,),
        in_specs=[
            pl.BlockSpec((1, 1, _R), lambda i: (i, 0, 0), memory_space=pltpu.SMEM),
            pl.BlockSpec((_R, 3), lambda i: (i, 0)),
            pl.BlockSpec((_R, _FP), lambda i: (i, 0)),
            full2(a1), full2(av1), full2(a2), full2(av2),
            full2(b1r), full2(b1f), full2(bb1),
        ],
        out_specs=pl.BlockSpec((c_out, w50), lambda i: (0, 0)),
        out_shape=jax.ShapeDtypeStruct((c_out, w50), jnp.float32),
        scratch_shapes=[pltpu.VMEM((_R, w50), jnp.float32)],
        interpret=interpret,
    )(seg.reshape(n // _R, 1, _R), x, fm, a1, av1, a2, av2, b1r, b1f, bb1)


def _enc3_kernel(seg_ref, x_ref, c50_ref, d1_ref, dv1_ref, d2_ref, dv2_ref,
                 e1r_ref, e1c_ref, ee1_ref, out_ref, row_ref):
    pid = pl.program_id(0)
    _init_fix(out_ref, pid)
    x = x_ref[...]
    r = jnp.maximum(_dot(x, d1_ref[...]) + dv1_ref[...], 0.0)
    r = jnp.maximum(_dot(r, d2_ref[...]) + dv2_ref[...], 0.0)
    c50 = c50_ref[...]
    e3 = jnp.maximum(_dot(r, e1r_ref[...]) + _dot(c50, e1c_ref[...])
                     + ee1_ref[...], 0.0)
    row_ref[...] = jnp.concatenate([e3, c50], axis=1)
    _seg_scatter_max(seg_ref, row_ref, out_ref, x.shape[0])
    _fixup(out_ref, pid)


def _enc3(x, c50, seg, d1, dv1, d2, dv2, e1r, e1c, ee1, c_out,
          interpret=False):
    n = x.shape[0]
    w50 = c50.shape[1]
    wout = e1r.shape[1] + w50
    full2 = lambda a: pl.BlockSpec(a.shape, lambda i: (0, 0))
    return pl.pallas_call(
        _enc3_kernel,
        grid=(n // _R,),
        in_specs=[
            pl.BlockSpec((1, 1, _R), lambda i: (i, 0, 0), memory_space=pltpu.SMEM),
            pl.BlockSpec((_R, 3), lambda i: (i, 0)),
            pl.BlockSpec((_R, w50), lambda i: (i, 0)),
            full2(d1), full2(dv1), full2(d2), full2(dv2),
            full2(e1r), full2(e1c), full2(ee1),
        ],
        out_specs=pl.BlockSpec((c_out, wout), lambda i: (0, 0)),
        out_shape=jax.ShapeDtypeStruct((c_out, wout), jnp.float32),
        scratch_shapes=[pltpu.VMEM((_R, wout), jnp.float32)],
        interpret=interpret,
    )(seg.reshape(n // _R, 1, _R), x, c50, d1, dv1, d2, dv2, e1r, e1c, ee1)


def kernel(relatives, relatives2, relatives3, W1, b1, W2, b2, W3, b3,
           A1, a1, A2, a2, B1, bb1, D1, d1, D2, d2, E1, ee1,
           cluster, indices2, cluster2, cluster3):
    row = lambda v: v.reshape(1, -1)
    c2 = relatives3.shape[0]
    feats1p = _enc1(relatives, cluster, W1, row(b1), W2, row(b2), W3,
                    row(b3), _C1)
    fm = _gather_rows(feats1p, indices2)
    out50 = _enc2(relatives2, fm, cluster2, A1, row(a1), A2, row(a2),
                  B1[:_F1], B1[_F1:], row(bb1), c2)
    out = _enc3(relatives3, out50, cluster3, D1, row(d1), D2, row(d2),
                E1[:_F1], E1[_F1:], row(ee1), _C3)
    return out


# speedup vs baseline: 2.0325x; 2.0325x over previous
import functools

import jax
import jax.numpy as jnp
from jax import lax
from jax.experimental import pallas as pl
from jax.experimental.pallas import tpu as pltpu
from jax.experimental.pallas import tpu_sc as plsc

_SENT = -3.0e38
_R = 1000
_FP = 32
_CH = 1000
_C1 = 50000
_C3 = 5000
_F1 = 20


def _seg_scatter_max(seg_ref, rows_ref, out_ref, nrows):
    def body(i, carry):
        s = seg_ref[0, 0, i]
        out_ref[pl.ds(s, 1), :] = jnp.maximum(
            out_ref[pl.ds(s, 1), :], rows_ref[pl.ds(i, 1), :])
        return carry

    lax.fori_loop(0, nrows, body, 0, unroll=8)


def _init_fix(out_ref, pid):
    @pl.when(pid == 0)
    def _():
        out_ref[...] = jnp.full(out_ref.shape, _SENT, out_ref.dtype)


def _fixup(out_ref, pid):
    @pl.when(pid == pl.num_programs(0) - 1)
    def _():
        o = out_ref[...]
        out_ref[...] = jnp.where(o == _SENT, 0.0, o)


def _dot(a, b):
    return jnp.dot(a, b, preferred_element_type=jnp.float32)


def _enc1_kernel(seg_ref, x_ref, w1_ref, b1_ref, w2_ref, b2_ref, w3_ref,
                 b3_ref, out_ref, h_ref):
    pid = pl.program_id(0)
    _init_fix(out_ref, pid)
    x = x_ref[...]
    h = jnp.maximum(_dot(x, w1_ref[...]) + b1_ref[...], 0.0)
    h = jnp.maximum(_dot(h, w2_ref[...]) + b2_ref[...], 0.0)
    h = _dot(h, w3_ref[...]) + b3_ref[...]
    pad = jnp.zeros((h.shape[0], _FP - h.shape[1]), h.dtype)
    h_ref[...] = jnp.concatenate([h, pad], axis=1)
    _seg_scatter_max(seg_ref, h_ref, out_ref, x.shape[0])
    _fixup(out_ref, pid)


def _enc1(x, seg, w1, b1, w2, b2, w3, b3, c_out, interpret=False):
    n = x.shape[0]
    full2 = lambda a: pl.BlockSpec(a.shape, lambda i: (0, 0))
    return pl.pallas_call(
        _enc1_kernel,
        grid=(n // _R,),
        in_specs=[
            pl.BlockSpec((1, 1, _R), lambda i: (i, 0, 0), memory_space=pltpu.SMEM),
            pl.BlockSpec((_R, 3), lambda i: (i, 0)),
            full2(w1), full2(b1), full2(w2), full2(b2), full2(w3), full2(b3),
        ],
        out_specs=pl.BlockSpec((c_out, _FP), lambda i: (0, 0)),
        out_shape=jax.ShapeDtypeStruct((c_out, _FP), jnp.float32),
        scratch_shapes=[pltpu.VMEM((_R, _FP), jnp.float32)],
        interpret=interpret,
    )(seg.reshape(n // _R, 1, _R), x, w1, b1, w2, b2, w3, b3)


def _gather_rows(table, idx):
    n = idx.shape[0]
    info = plsc.get_sparse_core_info()
    nw = info.num_cores * info.num_subcores
    bpw = n // nw
    nch = bpw // _CH
    mesh = plsc.VectorSubcoreMesh(core_axis_name="c", subcore_axis_name="s")

    @functools.partial(
        pl.kernel,
        mesh=mesh,
        compiler_params=pltpu.CompilerParams(use_tc_tiling_on_sc=False),
        out_type=jax.ShapeDtypeStruct((n, _FP), jnp.float32),
        scratch_types=[
            pltpu.VMEM((_CH,), jnp.int32),
            pltpu.VMEM((_CH, _FP), jnp.float32),
            pltpu.SemaphoreType.DMA,
        ],
    )
    def k(table_hbm, idx_hbm, out_hbm, idx_v, rows_v, sem):
        wid = lax.axis_index("s") * info.num_cores + lax.axis_index("c")
        base = wid * bpw

        def body(ci, carry):
            off = base + ci * _CH
            pltpu.sync_copy(idx_hbm.at[pl.ds(off, _CH)], idx_v)
            pltpu.async_copy(table_hbm.at[idx_v], rows_v, sem).wait()
            pltpu.sync_copy(rows_v, out_hbm.at[pl.ds(off, _CH)])
            return carry

        lax.fori_loop(0, nch, body, 0)

    return k(table, idx)


def _enc2_kernel(seg_ref, x_ref, fm_ref, a1_ref, av1_ref, a2_ref, av2_ref,
                 b1r_ref, b1f_ref, bb1_ref, out_ref, row_ref):
    pid = pl.program_id(0)
    _init_fix(out_ref, pid)
    x = x_ref[...]
    r = jnp.maximum(_dot(x, a1_ref[...]) + av1_ref[...], 0.0)
    r = jnp.maximum(_dot(r, a2_ref[...]) + av2_ref[...], 0.0)
    fm = fm_ref[...][:, :_F1]
    e = jnp.maximum(_dot(r, b1r_ref[...]) + _dot(fm, b1f_ref[...])
                    + bb1_ref[...], 0.0)
    row_ref[...] = jnp.concatenate([e, fm], axis=1)
    _seg_scatter_max(seg_ref, row_ref, out_ref, x.shape[0])
    _fixup(out_ref, pid)


def _enc2(x, fm, seg, a1, av1, a2, av2, b1r, b1f, bb1, c_out,
          interpret=False):
    n = x.shape[0]
    w50 = b1r.shape[1] + _F1
    full2 = lambda a: pl.BlockSpec(a.shape, lambda i: (0, 0))
    return pl.pallas_call(
        _enc2_kernel,
        grid=(n // _R,),
        in_specs=[
            pl.BlockSpec((1, 1, _R), lambda i: (i, 0, 0), memory_space=pltpu.SMEM),
            pl.BlockSpec((_R, 3), lambda i: (i, 0)),
            pl.BlockSpec((_R, _FP), lambda i: (i, 0)),
            full2(a1), full2(av1), full2(a2), full2(av2),
            full2(b1r), full2(b1f), full2(bb1),
        ],
        out_specs=pl.BlockSpec((c_out, w50), lambda i: (0, 0)),
        out_shape=jax.ShapeDtypeStruct((c_out, w50), jnp.float32),
        scratch_shapes=[pltpu.VMEM((_R, w50), jnp.float32)],
        interpret=interpret,
    )(seg.reshape(n // _R, 1, _R), x, fm, a1, av1, a2, av2, b1r, b1f, bb1)


def _enc3_kernel(seg_ref, x_ref, c50_ref, d1_ref, dv1_ref, d2_ref, dv2_ref,
                 e1r_ref, e1c_ref, ee1_ref, out_ref, row_ref):
    pid = pl.program_id(0)
    _init_fix(out_ref, pid)
    x = x_ref[...]
    r = jnp.maximum(_dot(x, d1_ref[...]) + dv1_ref[...], 0.0)
    r = jnp.maximum(_dot(r, d2_ref[...]) + dv2_ref[...], 0.0)
    c50 = c50_ref[...]
    e3 = jnp.maximum(_dot(r, e1r_ref[...]) + _dot(c50, e1c_ref[...])
                     + ee1_ref[...], 0.0)
    row_ref[...] = jnp.concatenate([e3, c50], axis=1)
    _seg_scatter_max(seg_ref, row_ref, out_ref, x.shape[0])
    _fixup(out_ref, pid)


def _enc3(x, c50, seg, d1, dv1, d2, dv2, e1r, e1c, ee1, c_out,
          interpret=False):
    n = x.shape[0]
    w50 = c50.shape[1]
    wout = e1r.shape[1] + w50
    full2 = lambda a: pl.BlockSpec(a.shape, lambda i: (0, 0))
    return pl.pallas_call(
        _enc3_kernel,
        grid=(n // _R,),
        in_specs=[
            pl.BlockSpec((1, 1, _R), lambda i: (i, 0, 0), memory_space=pltpu.SMEM),
            pl.BlockSpec((_R, 3), lambda i: (i, 0)),
            pl.BlockSpec((_R, w50), lambda i: (i, 0)),
            full2(d1), full2(dv1), full2(d2), full2(dv2),
            full2(e1r), full2(e1c), full2(ee1),
        ],
        out_specs=pl.BlockSpec((c_out, wout), lambda i: (0, 0)),
        out_shape=jax.ShapeDtypeStruct((c_out, wout), jnp.float32),
        scratch_shapes=[pltpu.VMEM((_R, wout), jnp.float32)],
        interpret=interpret,
    )(seg.reshape(n // _R, 1, _R), x, c50, d1, dv1, d2, dv2, e1r, e1c, ee1)


def kernel(relatives, relatives2, relatives3, W1, b1, W2, b2, W3, b3,
           A1, a1, A2, a2, B1, bb1, D1, d1, D2, d2, E1, ee1,
           cluster, indices2, cluster2, cluster3):
    row = lambda v: v.reshape(1, -1)
    c2 = relatives3.shape[0]
    feats1p = _enc1(relatives, cluster, W1, row(b1), W2, row(b2), W3,
                    row(b3), _C1)
    fm = _gather_rows(feats1p, indices2)
    out50 = _enc2(relatives2, fm, cluster2, A1, row(a1), A2, row(a2),
                  B1[:_F1], B1[_F1:], row(bb1), c2)
    out = _enc3(relatives3, out50, cluster3, D1, row(d1), D2, row(d2),
                E1[:_F1], E1[_F1:], row(ee1), _C3)
    return out


# speedup vs baseline: 5.2783x; 2.5969x over previous
import functools

import jax
import jax.numpy as jnp
from jax import lax
from jax.experimental import pallas as pl
from jax.experimental.pallas import tpu as pltpu
from jax.experimental.pallas import tpu_sc as plsc

_SENT = -3.0e38
_R = 1000
_FP = 32
_W50 = 64
_CH = 1000
_SCH = 512
_C1 = 50000
_C3 = 5000
_F1 = 20


def _dot(a, b):
    return jnp.dot(a, b, preferred_element_type=jnp.float32)


def _mlp1_kernel(x_ref, w1_ref, b1_ref, w2_ref, b2_ref, w3_ref, b3_ref,
                 out_ref):
    x = x_ref[...]
    h = jnp.maximum(_dot(x, w1_ref[...]) + b1_ref[...], 0.0)
    h = jnp.maximum(_dot(h, w2_ref[...]) + b2_ref[...], 0.0)
    h = _dot(h, w3_ref[...]) + b3_ref[...]
    pad = jnp.zeros((h.shape[0], _FP - h.shape[1]), h.dtype)
    out_ref[...] = jnp.concatenate([h, pad], axis=1)


def _mlp1(x, w1, b1, w2, b2, w3, b3):
    n = x.shape[0]
    full2 = lambda a: pl.BlockSpec(a.shape, lambda i: (0, 0))
    return pl.pallas_call(
        _mlp1_kernel,
        grid=(n // _R,),
        in_specs=[
            pl.BlockSpec((_R, 3), lambda i: (i, 0)),
            full2(w1), full2(b1), full2(w2), full2(b2), full2(w3), full2(b3),
        ],
        out_specs=pl.BlockSpec((_R, _FP), lambda i: (i, 0)),
        out_shape=jax.ShapeDtypeStruct((n, _FP), jnp.float32),
    )(x, w1, b1, w2, b2, w3, b3)


def _mlp2_kernel(x_ref, fm_ref, a1_ref, av1_ref, a2_ref, av2_ref,
                 b1r_ref, b1f_ref, bb1_ref, out_ref):
    x = x_ref[...]
    r = jnp.maximum(_dot(x, a1_ref[...]) + av1_ref[...], 0.0)
    r = jnp.maximum(_dot(r, a2_ref[...]) + av2_ref[...], 0.0)
    fm = fm_ref[...][:, :_F1]
    e = jnp.maximum(_dot(r, b1r_ref[...]) + _dot(fm, b1f_ref[...])
                    + bb1_ref[...], 0.0)
    pad = jnp.zeros((e.shape[0], _W50 - e.shape[1] - _F1), e.dtype)
    out_ref[...] = jnp.concatenate([e, fm, pad], axis=1)


def _mlp2(x, fm, a1, av1, a2, av2, b1r, b1f, bb1):
    n = x.shape[0]
    full2 = lambda a: pl.BlockSpec(a.shape, lambda i: (0, 0))
    return pl.pallas_call(
        _mlp2_kernel,
        grid=(n // _R,),
        in_specs=[
            pl.BlockSpec((_R, 3), lambda i: (i, 0)),
            pl.BlockSpec((_R, _FP), lambda i: (i, 0)),
            full2(a1), full2(av1), full2(a2), full2(av2),
            full2(b1r), full2(b1f), full2(bb1),
        ],
        out_specs=pl.BlockSpec((_R, _W50), lambda i: (i, 0)),
        out_shape=jax.ShapeDtypeStruct((n, _W50), jnp.float32),
    )(x, fm, a1, av1, a2, av2, b1r, b1f, bb1)


def _bisect(seg_hbm, probe, target, n):
    def body(t, c):
        lo, hi = c
        active = lo < hi
        mid = jnp.minimum((lo + hi) // 2, n - 1)
        mal = (mid // 16) * 16
        pltpu.sync_copy(seg_hbm.at[pl.ds(mal, 16)], probe.at[pl.ds(0, 16)])
        v = probe[pl.ds(mid - mal, 16)][0]
        ge = v >= target
        new_lo = jnp.where(jnp.logical_and(active, jnp.logical_not(ge)),
                           mid + 1, lo)
        new_hi = jnp.where(jnp.logical_and(active, ge), mid, hi)
        return (new_lo, new_hi)

    lo, _ = lax.fori_loop(0, 20, body, (jnp.int32(0), jnp.int32(n)))
    return lo


def _segmax_sc(rows, seg, c_total, w):
    n = rows.shape[0]
    info = plsc.get_sparse_core_info()
    nw = info.num_cores * info.num_subcores
    spt = -(-c_total // nw)
    cpad = spt * nw
    mesh = plsc.VectorSubcoreMesh(core_axis_name="c", subcore_axis_name="s")

    @functools.partial(
        pl.kernel,
        mesh=mesh,
        compiler_params=pltpu.CompilerParams(use_tc_tiling_on_sc=False),
        out_type=jax.ShapeDtypeStruct((cpad, w), jnp.float32),
        scratch_types=[
            pltpu.VMEM((_SCH + 16,), jnp.int32),
            pltpu.VMEM((_SCH, w), jnp.float32),
            pltpu.VMEM((spt + 8, w), jnp.float32),
            pltpu.VMEM((32,), jnp.int32),
        ],
    )
    def k(rows_hbm, seg_hbm, out_hbm, segv, rowsv, buf, probe):
        wid = lax.axis_index("s") * info.num_cores + lax.axis_index("c")
        my_lo = wid * spt
        my_hi = my_lo + spt

        def initrow(j, c):
            for t in range(w // 16):
                buf[j, pl.ds(t * 16, 16)] = jnp.full((16,), _SENT, jnp.float32)
            return c

        lax.fori_loop(0, spt + 8, initrow, 0)

        lo = _bisect(seg_hbm, probe, my_lo, n)
        hi = _bisect(seg_hbm, probe, my_hi, n)
        lo_al = (lo // 8) * 8
        nch = (hi - lo_al + _SCH - 1) // _SCH

        def chunk(ci, c):
            start = jnp.minimum(lo_al + ci * _SCH, n - _SCH)
            pltpu.sync_copy(seg_hbm.at[pl.ds(start, _SCH)],
                            segv.at[pl.ds(0, _SCH)])
            pltpu.sync_copy(rows_hbm.at[pl.ds(start, _SCH)], rowsv)

            def rowi(i, cc):
                s0 = segv[pl.ds(i, 16)][0]
                inr = jnp.logical_and(s0 >= my_lo, s0 < my_hi)
                j = jnp.where(inr, s0 - my_lo, spt)
                for t in range(w // 16):
                    sl = pl.ds(t * 16, 16)
                    buf[j, sl] = jnp.maximum(buf[j, sl], rowsv[i, sl])
                return cc

            lax.fori_loop(0, _SCH, rowi, 0)
            return c

        lax.fori_loop(0, nch, chunk, 0)

        def fixrow(j, c):
            for t in range(w // 16):
                sl = pl.ds(t * 16, 16)
                v = buf[j, sl]
                buf[j, sl] = jnp.where(v == _SENT, 0.0, v)
            return c

        lax.fori_loop(0, spt, fixrow, 0)
        pltpu.sync_copy(buf.at[pl.ds(0, spt)], out_hbm.at[pl.ds(my_lo, spt)])

    return k(rows, seg)


def _gather_rows(table, idx):
    n = idx.shape[0]
    info = plsc.get_sparse_core_info()
    nw = info.num_cores * info.num_subcores
    bpw = n // nw
    nch = bpw // _CH
    mesh = plsc.VectorSubcoreMesh(core_axis_name="c", subcore_axis_name="s")

    @functools.partial(
        pl.kernel,
        mesh=mesh,
        compiler_params=pltpu.CompilerParams(use_tc_tiling_on_sc=False),
        out_type=jax.ShapeDtypeStruct((n, _FP), jnp.float32),
        scratch_types=[
            pltpu.VMEM((_CH,), jnp.int32),
            pltpu.VMEM((_CH, _FP), jnp.float32),
            pltpu.SemaphoreType.DMA,
        ],
    )
    def k(table_hbm, idx_hbm, out_hbm, idx_v, rows_v, sem):
        wid = lax.axis_index("s") * info.num_cores + lax.axis_index("c")
        base = wid * bpw

        def body(ci, carry):
            off = base + ci * _CH
            pltpu.sync_copy(idx_hbm.at[pl.ds(off, _CH)], idx_v)
            pltpu.async_copy(table_hbm.at[idx_v], rows_v, sem).wait()
            pltpu.sync_copy(rows_v, out_hbm.at[pl.ds(off, _CH)])
            return carry

        lax.fori_loop(0, nch, body, 0)

    return k(table, idx)


def _enc3_kernel(seg_ref, x_ref, c50_ref, d1_ref, dv1_ref, d2_ref, dv2_ref,
                 e1r_ref, e1c_ref, ee1_ref, out_ref, row_ref):
    pid = pl.program_id(0)

    @pl.when(pid == 0)
    def _():
        out_ref[...] = jnp.full(out_ref.shape, _SENT, out_ref.dtype)

    x = x_ref[...]
    r = jnp.maximum(_dot(x, d1_ref[...]) + dv1_ref[...], 0.0)
    r = jnp.maximum(_dot(r, d2_ref[...]) + dv2_ref[...], 0.0)
    c50 = c50_ref[...][:, :_F1 + 30]
    e3 = jnp.maximum(_dot(r, e1r_ref[...]) + _dot(c50, e1c_ref[...])
                     + ee1_ref[...], 0.0)
    row_ref[...] = jnp.concatenate([e3, c50], axis=1)

    def body(i, carry):
        s = seg_ref[0, 0, i]
        out_ref[pl.ds(s, 1), :] = jnp.maximum(
            out_ref[pl.ds(s, 1), :], row_ref[pl.ds(i, 1), :])
        return carry

    lax.fori_loop(0, x.shape[0], body, 0, unroll=8)

    @pl.when(pid == pl.num_programs(0) - 1)
    def _():
        o = out_ref[...]
        out_ref[...] = jnp.where(o == _SENT, 0.0, o)


def _enc3(x, c50, seg, d1, dv1, d2, dv2, e1r, e1c, ee1, c_out):
    n = x.shape[0]
    wout = e1r.shape[1] + _F1 + 30
    full2 = lambda a: pl.BlockSpec(a.shape, lambda i: (0, 0))
    return pl.pallas_call(
        _enc3_kernel,
        grid=(n // _R,),
        in_specs=[
            pl.BlockSpec((1, 1, _R), lambda i: (i, 0, 0),
                         memory_space=pltpu.SMEM),
            pl.BlockSpec((_R, 3), lambda i: (i, 0)),
            pl.BlockSpec((_R, _W50), lambda i: (i, 0)),
            full2(d1), full2(dv1), full2(d2), full2(dv2),
            full2(e1r), full2(e1c), full2(ee1),
        ],
        out_specs=pl.BlockSpec((c_out, wout), lambda i: (0, 0)),
        out_shape=jax.ShapeDtypeStruct((c_out, wout), jnp.float32),
        scratch_shapes=[pltpu.VMEM((_R, wout), jnp.float32)],
    )(seg.reshape(n // _R, 1, _R), x, c50, d1, dv1, d2, dv2, e1r, e1c, ee1)


def kernel(relatives, relatives2, relatives3, W1, b1, W2, b2, W3, b3,
           A1, a1, A2, a2, B1, bb1, D1, d1, D2, d2, E1, ee1,
           cluster, indices2, cluster2, cluster3):
    row = lambda v: v.reshape(1, -1)
    c2 = relatives3.shape[0]
    h = _mlp1(relatives, W1, row(b1), W2, row(b2), W3, row(b3))
    feats1p = _segmax_sc(h, cluster, _C1, _FP)
    fm = _gather_rows(feats1p, indices2)
    rows50 = _mlp2(relatives2, fm, A1, row(a1), A2, row(a2),
                   B1[:_F1], B1[_F1:], row(bb1))
    out50 = _segmax_sc(rows50, cluster2, c2, _W50)
    out = _enc3(relatives3, out50, cluster3, D1, row(d1), D2, row(d2),
                E1[:_F1], E1[_F1:], row(ee1), _C3)
    return out
